# baseline (speedup 1.0000x reference)
import functools
import math

import jax
import jax.numpy as jnp
from jax import lax
from jax.experimental import pallas as pl
from jax.experimental.pallas import tpu as pltpu
from jax.experimental.pallas import tpu_sc as plsc

_SIZE = 32000
_SMOOTHING = 0.1
_CONF = 1.0 - _SMOOTHING
_EPS = _SMOOTHING / (_SIZE - 2)
_N = 2048
_CBLK = 3200

_NC = 2
_NS = 16
_NW = _NC * _NS
_BPW = _N // _NW
_LANES = 16


def _dense_kernel(tgt_ref, pred_ref, out_ref):
    k = pl.program_id(0)
    eps = jnp.float32(_EPS)

    t = tgt_ref[...]
    t_is_zero = t == 0
    any_z = jnp.any(t_is_zero)
    any_nz = jnp.any(jnp.logical_not(t_is_zero))
    rid = jax.lax.broadcasted_iota(jnp.int32, (_N, 1), 0)
    w = jnp.where((rid == 0) & any_nz, 0.0, 1.0) * jnp.where(
        (rid == 1) & any_z, 0.0, 1.0
    )

    rowsum = jnp.sum(pred_ref[...], axis=1, keepdims=True)
    partial = jnp.sum(w * (-eps) * rowsum)

    @pl.when(k == 0)
    def _init():
        out_ref[...] = jnp.zeros((1, 1), jnp.float32)

    out_ref[...] += partial.reshape(1, 1)


def _sparse_kernel(pred_flat, tgt, out, tgt_v, idx_v, vals_v, acc_v, sem):
    wid = lax.axis_index("s") * _NC + lax.axis_index("c")
    base = wid * _BPW

    eps = jnp.float32(_EPS)
    conf = jnp.float32(_CONF)
    a_zero = jnp.float32((_SIZE - 1) * _EPS * math.log(_EPS))
    a_nonzero = jnp.float32(
        (_SIZE - 2) * _EPS * math.log(_EPS) + _CONF * math.log(_CONF)
    )

    pltpu.sync_copy(tgt, tgt_v)

    lane = lax.iota(jnp.int32, _LANES)

    for j in range(_BPW // _LANES):
        rows = base + j * _LANES + lane
        t_chunk = tgt_v[pl.ds(base + j * _LANES, _LANES)]
        idx_v[pl.ds(j * _LANES, _LANES)] = rows * _SIZE + t_chunk
        idx_v[pl.ds(_BPW + j * _LANES, _LANES)] = rows * _SIZE

    pltpu.async_copy(pred_flat.at[idx_v], vals_v, sem).wait()

    def _count(i, nz):
        chunk = tgt_v[pl.ds(i * _LANES, _LANES)]
        return nz + plsc.all_reduce_population_count(chunk == 0)

    num_zero = lax.fori_loop(
        0, _N // _LANES, _count, jnp.zeros((_LANES,), jnp.int32)
    )
    zero_row0 = jnp.where(num_zero < _N, jnp.float32(1.0), jnp.float32(0.0))
    zero_row1 = jnp.where(num_zero > 0, jnp.float32(1.0), jnp.float32(0.0))

    acc = jnp.zeros((_LANES,), jnp.float32)
    for j in range(_BPW // _LANES):
        rows = base + j * _LANES + lane
        t_chunk = tgt_v[pl.ds(base + j * _LANES, _LANES)]
        pt = vals_v[pl.ds(j * _LANES, _LANES)]
        p0 = vals_v[pl.ds(_BPW + j * _LANES, _LANES)]
        tz = t_chunk == 0
        a_i = jnp.where(tz, a_zero, a_nonzero)
        beta = jnp.where(tz, jnp.float32(0.0), eps - conf)
        w = (
            jnp.float32(1.0)
            - jnp.where(rows == 0, zero_row0, jnp.float32(0.0))
            - jnp.where(rows == 1, zero_row1, jnp.float32(0.0))
        )
        acc = acc + w * (a_i + eps * p0 + beta * pt)

    acc_v[...] = acc
    pltpu.sync_copy(acc_v, out.at[wid])


@functools.partial(jax.jit, static_argnames=("interpret",))
def kernel(prediction, target, interpret=False):
    n, size = prediction.shape
    tgt_i32 = target.astype(jnp.int32)

    dense = pl.pallas_call(
        _dense_kernel,
        grid=(size // _CBLK,),
        in_specs=[
            pl.BlockSpec((n, 1), lambda k: (0, 0)),
            pl.BlockSpec((n, _CBLK), lambda k: (0, k)),
        ],
        out_specs=pl.BlockSpec((1, 1), lambda k: (0, 0)),
        out_shape=jax.ShapeDtypeStruct((1, 1), jnp.float32),
        interpret=interpret,
    )(tgt_i32.reshape(n, 1), prediction)

    sparse_fn = pl.kernel(
        _sparse_kernel,
        mesh=plsc.VectorSubcoreMesh(core_axis_name="c", subcore_axis_name="s"),
        out_type=jax.ShapeDtypeStruct((_NW, _LANES), jnp.float32),
        compiler_params=pltpu.CompilerParams(needs_layout_passes=False),
        scratch_types=[
            pltpu.VMEM((_N,), jnp.int32),
            pltpu.VMEM((2 * _BPW,), jnp.int32),
            pltpu.VMEM((2 * _BPW,), jnp.float32),
            pltpu.VMEM((_LANES,), jnp.float32),
            pltpu.SemaphoreType.DMA,
        ],
    )
    sparse = sparse_fn(prediction.reshape(-1), tgt_i32)

    return dense[0, 0] + jnp.sum(sparse)


# speedup vs baseline: 2.7128x; 2.7128x over previous
import functools
import math

import jax
import jax.numpy as jnp
from jax import lax
from jax.experimental import pallas as pl
from jax.experimental.pallas import tpu as pltpu
from jax.experimental.pallas import tpu_sc as plsc

_SIZE = 32000
_SMOOTHING = 0.1
_CONF = 1.0 - _SMOOTHING
_EPS = _SMOOTHING / (_SIZE - 2)
_N = 2048
_CBLK = 3200

_NC = 2
_NS = 16
_NW = _NC * _NS
_BPW = _N // _NW
_LANES = 16

_A_ZERO = (_SIZE - 1) * _EPS * math.log(_EPS)
_A_NONZERO = (_SIZE - 2) * _EPS * math.log(_EPS) + _CONF * math.log(_CONF)


def _dense_kernel(tgt_ref, pred_ref, out_ref):
    k = pl.program_id(0)
    eps = jnp.float32(_EPS)
    conf = jnp.float32(_CONF)

    t = tgt_ref[...]
    t_is_zero = t == 0
    any_z = jnp.any(t_is_zero)
    any_nz = jnp.any(jnp.logical_not(t_is_zero))
    rid = jax.lax.broadcasted_iota(jnp.int32, (_N, 1), 0)
    w = jnp.where((rid == 0) & any_nz, 0.0, 1.0) * jnp.where(
        (rid == 1) & any_z, 0.0, 1.0
    )

    block = pred_ref[...]
    col = jax.lax.broadcasted_iota(jnp.int32, (_N, _CBLK), 1) + k * _CBLK
    sel = jnp.where(col == t, block, 0.0)
    rowsum = jnp.sum(block, axis=1, keepdims=True)
    psel = jnp.sum(sel, axis=1, keepdims=True)

    beta = jnp.where(t_is_zero, 0.0, eps - conf)
    partial = jnp.sum(w * (beta * psel - eps * rowsum))

    @pl.when(k == 0)
    def _init():
        p0 = block[:, 0:1]
        out_ref[...] = jnp.sum(w * eps * p0).reshape(1, 1)

    out_ref[...] += partial.reshape(1, 1)


def _sparse_kernel(tgt, out, tgt_v, acc_v):
    wid = lax.axis_index("s") * _NC + lax.axis_index("c")
    base = wid * _BPW

    pltpu.sync_copy(tgt, tgt_v)

    def _count(i, nz):
        chunk = tgt_v[pl.ds(i * _LANES, _LANES)]
        return nz + plsc.all_reduce_population_count(chunk == 0)

    num_zero = lax.fori_loop(
        0, _N // _LANES, _count, jnp.zeros((_LANES,), jnp.int32)
    )
    zero_row0 = jnp.where(num_zero < _N, jnp.float32(1.0), jnp.float32(0.0))
    zero_row1 = jnp.where(num_zero > 0, jnp.float32(1.0), jnp.float32(0.0))

    lane = lax.iota(jnp.int32, _LANES)
    acc = jnp.zeros((_LANES,), jnp.float32)
    for j in range(_BPW // _LANES):
        rows = base + j * _LANES + lane
        t_chunk = tgt_v[pl.ds(base + j * _LANES, _LANES)]
        a_i = jnp.where(
            t_chunk == 0, jnp.float32(_A_ZERO), jnp.float32(_A_NONZERO)
        )
        w = (
            jnp.float32(1.0)
            - jnp.where(rows == 0, zero_row0, jnp.float32(0.0))
            - jnp.where(rows == 1, zero_row1, jnp.float32(0.0))
        )
        acc = acc + w * a_i

    acc_v[...] = acc
    pltpu.sync_copy(acc_v, out.at[wid])


@functools.partial(jax.jit, static_argnames=("interpret",))
def kernel(prediction, target, interpret=False):
    n, size = prediction.shape
    tgt_i32 = target.astype(jnp.int32)

    dense = pl.pallas_call(
        _dense_kernel,
        grid=(size // _CBLK,),
        in_specs=[
            pl.BlockSpec((n, 1), lambda k: (0, 0)),
            pl.BlockSpec((n, _CBLK), lambda k: (0, k)),
        ],
        out_specs=pl.BlockSpec((1, 1), lambda k: (0, 0)),
        out_shape=jax.ShapeDtypeStruct((1, 1), jnp.float32),
        interpret=interpret,
    )(tgt_i32.reshape(n, 1), prediction)

    sparse_fn = pl.kernel(
        _sparse_kernel,
        mesh=plsc.VectorSubcoreMesh(core_axis_name="c", subcore_axis_name="s"),
        out_type=jax.ShapeDtypeStruct((_NW, _LANES), jnp.float32),
        compiler_params=pltpu.CompilerParams(needs_layout_passes=False),
        scratch_types=[
            pltpu.VMEM((_N,), jnp.int32),
            pltpu.VMEM((_LANES,), jnp.float32),
        ],
    )
    sparse = sparse_fn(tgt_i32)

    return dense[0, 0] + jnp.sum(sparse)
